# baseline (speedup 1.0000x reference)
import functools

import jax
import jax.numpy as jnp
from jax import lax
from jax.experimental import pallas as pl
from jax.experimental.pallas import tpu as pltpu
from jax.experimental.pallas import tpu_sc as plsc

N = 10000
D = 128
H = 128
C = 64
E = 320000

NC = 2
NS = 16
NW = NC * NS
CHUNK = 128
CPT = 80
CPT0 = 80
CPT1 = 80
CPTX = max(CPT0, CPT1)
TOTCH = NS * (CPT0 + CPT1)
EPAD = TOTCH * CHUNK
STAGE_PAD = CPTX
ROWS_PER_TILE = 640
NPAD = NS * ROWS_PER_TILE
NBUF = 2
NIB = 4
UNROLL = 4

_mesh = plsc.VectorSubcoreMesh(core_axis_name="c", subcore_axis_name="s")


@functools.partial(
    pl.kernel,
    out_type=jax.ShapeDtypeStruct((NC * NPAD,), jnp.float32),
    mesh=_mesh,
    scratch_types=[
        pltpu.VMEM((CPT, CHUNK), jnp.int32),
        pltpu.VMEM((CHUNK,), jnp.float32),
        pltpu.VMEM_SHARED((NPAD,), jnp.float32),
    ],
)
def _deg_kernel(col_hbm, zeros_hbm, deg_out, col_v, ones_v, deg_sh):
    cid = lax.axis_index("c")
    sid = lax.axis_index("s")
    wid = cid * NS + sid

    base = sid * ROWS_PER_TILE
    pltpu.sync_copy(zeros_hbm.at[pl.ds(base, ROWS_PER_TILE)],
                    deg_sh.at[pl.ds(base, ROWS_PER_TILE)])

    for i in range(CHUNK // 16):
        ones_v[pl.ds(i * 16, 16)] = jnp.ones((16,), jnp.float32)

    pltpu.sync_copy(col_hbm.at[pl.ds(wid * CPT, CPT)], col_v)

    plsc.subcore_barrier()

    def body(j, carry):
        pltpu.sync_copy(ones_v, deg_sh.at[col_v.at[j]], add=True)
        return carry

    lax.fori_loop(0, CPT, body, 0)

    plsc.subcore_barrier()

    pltpu.sync_copy(deg_sh.at[pl.ds(base, ROWS_PER_TILE)],
                    deg_out.at[pl.ds(cid * NPAD + base, ROWS_PER_TILE)])


def _xs_body(deg_ref, x_ref, xs_ref):
    degsum = deg_ref[:, 0:1] + deg_ref[:, 1:2] + 1.0
    xs_ref[...] = x_ref[...] * lax.rsqrt(degsum)


def _xs_call(degT, x):
    blk = 1000
    return pl.pallas_call(
        _xs_body,
        grid=(N // blk,),
        in_specs=[
            pl.BlockSpec((blk, 2), lambda i: (i, 0)),
            pl.BlockSpec((blk, D), lambda i: (i, 0)),
        ],
        out_specs=pl.BlockSpec((blk, D), lambda i: (i, 0)),
        out_shape=jax.ShapeDtypeStruct((N, D), jnp.float32),
    )(degT, x)


@functools.partial(
    pl.kernel,
    out_type=jax.ShapeDtypeStruct((NC, NPAD, D), jnp.float32),
    mesh=_mesh,
    scratch_types=[
        pltpu.VMEM((NIB, CHUNK), jnp.int32),
        pltpu.VMEM((CPTX, CHUNK), jnp.int32),
        pltpu.VMEM((NBUF, CHUNK, D), jnp.float32),
        pltpu.VMEM_SHARED((NPAD, D), jnp.float32),
        pltpu.SemaphoreType.DMA((NIB,)),
        pltpu.SemaphoreType.DMA((NBUF,)),
        pltpu.SemaphoreType.DMA((NBUF,)),
    ],
)
def _scatter_kernel(xs_hbm, row_hbm, col_hbm, zeros_hbm, t_out,
                    row_v, col_v, buf, t_sh, isem, gsem, ssem):
    cid = lax.axis_index("c")
    sid = lax.axis_index("s")
    c_cpt = jnp.where(cid == 0, CPT0, CPT1)
    cbase = jnp.where(cid == 0, sid * CPT0, NS * CPT0 + sid * CPT1)

    base = sid * ROWS_PER_TILE
    with jax.named_scope("zero_init"):
        pltpu.sync_copy(zeros_hbm.at[pl.ds(base, ROWS_PER_TILE)],
                        t_sh.at[pl.ds(base, ROWS_PER_TILE)])

    with jax.named_scope("idx_stage"):
        pltpu.sync_copy(col_hbm.at[pl.ds(cbase, CPTX)], col_v)

    with jax.named_scope("pre_barrier"):
        plsc.subcore_barrier()

    def _ridx(j, s):
        return pltpu.make_async_copy(
            row_hbm.at[cbase + j], row_v.at[s], isem.at[s])

    def _gather(s, b):
        return pltpu.make_async_copy(
            xs_hbm.at[row_v.at[s]], buf.at[b], gsem.at[b])

    def _scatter_start(j, b):
        pltpu.async_copy(buf.at[b], t_sh.at[col_v.at[j]], ssem.at[b],
                         add=True)

    def _scatter_wait(j, b):
        pltpu.make_async_copy(
            buf.at[b], t_sh.at[col_v.at[j]], ssem.at[b]).wait()

    _ridx(0, 0).start()
    _ridx(1, 1).start()
    _ridx(0, 0).wait()
    _gather(0, 0).start()

    def body(i, carry):
        for b in range(UNROLL):
            j = UNROLL * i + b
            bb = b % NBUF
            _gather((b % NIB), bb).wait()
            _scatter_start(j, bb)

            @pl.when(j + 2 < c_cpt)
            def _():
                _ridx(j + 2, (b + 2) % NIB).start()

            @pl.when(j + 1 < c_cpt)
            def _():
                _ridx(j + 1, (b + 1) % NIB).wait()

            @pl.when(jnp.logical_and(j + 1 < c_cpt, j >= 1))
            def _():
                _scatter_wait(j - 1, (b + 1) % NBUF)

            @pl.when(j + 1 < c_cpt)
            def _():
                _gather((b + 1) % NIB, (b + 1) % NBUF).start()

        return carry

    with jax.named_scope("edge_loop"):
        lax.fori_loop(0, c_cpt // UNROLL, body, 0)

        for b in range(NBUF):
            _scatter_wait(c_cpt - NBUF + b, b)

    with jax.named_scope("post_barrier"):
        plsc.subcore_barrier()

    with jax.named_scope("writeout"):
        pltpu.sync_copy(t_sh.at[pl.ds(base, ROWS_PER_TILE)],
                        t_out.at[cid, pl.ds(base, ROWS_PER_TILE)])


def _elu(v):
    return jnp.maximum(v, 0.0) + (jnp.exp(jnp.minimum(v, 0.0)) - 1.0)


def _dense_body(t_ref, deg_ref, xs_ref, w0_ref, b0_ref, l0_ref, lb0_ref,
                w1_ref, b1_ref, l1_ref, lb1_ref, owa_ref, owb_ref, ob_ref,
                out_ref):
    dis = lax.rsqrt(deg_ref[:, 0:1] + deg_ref[:, 1:2] + 1.0)
    agg = (t_ref[0] + t_ref[1] + xs_ref[...]) * dis
    f32 = jnp.float32
    a0 = _elu(jnp.dot(agg, w0_ref[...], preferred_element_type=f32, precision=lax.Precision.HIGHEST) + b0_ref[...])
    y0 = jnp.dot(a0, l0_ref[...], preferred_element_type=f32, precision=lax.Precision.HIGHEST) + lb0_ref[...]
    a1 = _elu(jnp.dot(agg, w1_ref[...], preferred_element_type=f32, precision=lax.Precision.HIGHEST) + b1_ref[...])
    y1 = jnp.dot(a1, l1_ref[...], preferred_element_type=f32, precision=lax.Precision.HIGHEST) + lb1_ref[...]
    out_ref[...] = (jnp.dot(y0, owa_ref[...], preferred_element_type=f32, precision=lax.Precision.HIGHEST)
                    + jnp.dot(y1, owb_ref[...], preferred_element_type=f32, precision=lax.Precision.HIGHEST)
                    + ob_ref[...])


def _dense_call(t2, degT, xs, w0, b0, l0, lb0, w1, b1, l1, lb1, owa, owb, ob):
    blk = 1000

    def full(shape):
        return pl.BlockSpec(shape, lambda i, _s=shape: tuple(0 for _ in _s))

    return pl.pallas_call(
        _dense_body,
        grid=(N // blk,),
        in_specs=[
            pl.BlockSpec((NC, blk, D), lambda i: (0, i, 0)),
            pl.BlockSpec((blk, 2), lambda i: (i, 0)),
            pl.BlockSpec((blk, D), lambda i: (i, 0)),
            full((D, H)), full((1, H)), full((H, H)), full((1, H)),
            full((D, H)), full((1, H)), full((H, H)), full((1, H)),
            full((H, C)), full((H, C)), full((1, C)),
        ],
        out_specs=pl.BlockSpec((blk, C), lambda i: (i, 0)),
        out_shape=jax.ShapeDtypeStruct((N, C), jnp.float32),
    )(t2, degT, xs, w0, b0, l0, lb0, w1, b1, l1, lb1, owa, owb, ob)


def kernel(x, edge_index, conv0_W, conv0_b, lin0_W, lin0_b,
           conv1_W, conv1_b, lin1_W, lin1_b, out_W, out_b):
    pad = EPAD + STAGE_PAD * CHUNK - E
    nrows = TOTCH + STAGE_PAD
    row_p = jnp.concatenate(
        [edge_index[0], jnp.zeros((pad,), jnp.int32)]).reshape(nrows, CHUNK)
    col_p = jnp.concatenate(
        [edge_index[1], jnp.full((pad,), N, jnp.int32)]).reshape(nrows, CHUNK)

    deg2 = _deg_kernel(col_p, jnp.zeros((NPAD,), jnp.float32)).reshape(NC, NPAD)
    degT = deg2.T

    xs = _xs_call(degT[:N], x)

    t2 = _scatter_kernel(xs, row_p, col_p, jnp.zeros((NPAD, D), jnp.float32))

    return _dense_call(
        t2[:, :N], degT[:N], xs,
        conv0_W, conv0_b.reshape(1, H), lin0_W, lin0_b.reshape(1, H),
        conv1_W, conv1_b.reshape(1, H), lin1_W, lin1_b.reshape(1, H),
        out_W[:H], out_W[H:], out_b.reshape(1, C))


# speedup vs baseline: 2.3439x; 2.3439x over previous
import functools

import jax
import jax.numpy as jnp
from jax import lax
from jax.experimental import pallas as pl
from jax.experimental.pallas import tpu as pltpu
from jax.experimental.pallas import tpu_sc as plsc

N = 10000
D = 128
H = 128
C = 64
E = 320000

NC = 2
NS = 16
NW = NC * NS
CHUNK = 128
CPT = 80
CPT0 = 80
CPT1 = 80
CPTX = max(CPT0, CPT1)
TOTCH = NS * (CPT0 + CPT1)
EPAD = TOTCH * CHUNK
STAGE_PAD = CPTX
ROWS_PER_TILE = 640
NPAD = NS * ROWS_PER_TILE
NBUF = 2
NIB = 4
UNROLL = 4

_mesh = plsc.VectorSubcoreMesh(core_axis_name="c", subcore_axis_name="s")


@functools.partial(
    pl.kernel,
    out_type=jax.ShapeDtypeStruct((NC * NPAD,), jnp.float32),
    mesh=_mesh,
    scratch_types=[
        pltpu.VMEM((CPT, CHUNK), jnp.int32),
        pltpu.VMEM((CHUNK,), jnp.float32),
        pltpu.VMEM_SHARED((NPAD,), jnp.float32),
    ],
)
def _deg_kernel(col_hbm, zeros_hbm, deg_out, col_v, ones_v, deg_sh):
    cid = lax.axis_index("c")
    sid = lax.axis_index("s")
    wid = cid * NS + sid

    base = sid * ROWS_PER_TILE
    pltpu.sync_copy(zeros_hbm.at[pl.ds(base, ROWS_PER_TILE)],
                    deg_sh.at[pl.ds(base, ROWS_PER_TILE)])

    for i in range(CHUNK // 16):
        ones_v[pl.ds(i * 16, 16)] = jnp.ones((16,), jnp.float32)

    pltpu.sync_copy(col_hbm.at[pl.ds(wid * CPT, CPT)], col_v)

    plsc.subcore_barrier()

    def body(j, carry):
        pltpu.sync_copy(ones_v, deg_sh.at[col_v.at[j]], add=True)
        return carry

    lax.fori_loop(0, CPT, body, 0)

    plsc.subcore_barrier()

    pltpu.sync_copy(deg_sh.at[pl.ds(base, ROWS_PER_TILE)],
                    deg_out.at[pl.ds(cid * NPAD + base, ROWS_PER_TILE)])


def _xs_body(deg_ref, x_ref, xs_ref):
    degsum = deg_ref[:, 0:1] + deg_ref[:, 1:2] + 1.0
    xs_ref[...] = x_ref[...] * lax.rsqrt(degsum)


def _xs_call(degT, x):
    blk = 1000
    return pl.pallas_call(
        _xs_body,
        grid=(N // blk,),
        in_specs=[
            pl.BlockSpec((blk, 2), lambda i: (i, 0)),
            pl.BlockSpec((blk, D), lambda i: (i, 0)),
        ],
        out_specs=pl.BlockSpec((blk, D), lambda i: (i, 0)),
        out_shape=jax.ShapeDtypeStruct((N, D), jnp.float32),
    )(degT, x)


@functools.partial(
    pl.kernel,
    out_type=jax.ShapeDtypeStruct((NC, NPAD, D), jnp.float32),
    mesh=_mesh,
    scratch_types=[
        pltpu.VMEM((NIB, CHUNK), jnp.int32),
        pltpu.VMEM((CPTX, CHUNK), jnp.int32),
        pltpu.VMEM((NBUF, CHUNK, D), jnp.float32),
        pltpu.VMEM_SHARED((NPAD, D), jnp.float32),
        pltpu.SemaphoreType.DMA((NIB,)),
        pltpu.SemaphoreType.DMA((NBUF,)),
        pltpu.SemaphoreType.DMA((NBUF,)),
    ],
)
def _scatter_kernel(xs_hbm, row_hbm, col_hbm, zeros_hbm, t_out,
                    row_v, col_v, buf, t_sh, isem, gsem, ssem):
    cid = lax.axis_index("c")
    sid = lax.axis_index("s")
    c_cpt = jnp.where(cid == 0, CPT0, CPT1)
    cbase = jnp.where(cid == 0, sid * CPT0, NS * CPT0 + sid * CPT1)

    base = sid * ROWS_PER_TILE
    pltpu.sync_copy(zeros_hbm.at[pl.ds(base, ROWS_PER_TILE)],
                    t_sh.at[pl.ds(base, ROWS_PER_TILE)])

    pltpu.sync_copy(col_hbm.at[pl.ds(cbase, CPTX)], col_v)

    plsc.subcore_barrier()

    def _ridx(j, s):
        return pltpu.make_async_copy(
            row_hbm.at[cbase + j], row_v.at[s], isem.at[s])

    def _gather(s, b):
        return pltpu.make_async_copy(
            xs_hbm.at[row_v.at[s]], buf.at[b], gsem.at[b])

    def _scatter_start(j, b):
        pltpu.async_copy(buf.at[b], t_sh.at[col_v.at[j]], ssem.at[b],
                         add=True)

    def _scatter_wait(j, b):
        pltpu.make_async_copy(
            buf.at[b], t_sh.at[col_v.at[j]], ssem.at[b]).wait()

    _ridx(0, 0).start()
    _ridx(1, 1).start()
    _ridx(0, 0).wait()
    _gather(0, 0).start()

    def body(i, carry):
        for b in range(UNROLL):
            j = UNROLL * i + b
            bb = b % NBUF
            _gather((b % NIB), bb).wait()
            _scatter_start(j, bb)

            @pl.when(j + 2 < c_cpt)
            def _():
                _ridx(j + 2, (b + 2) % NIB).start()

            @pl.when(j + 1 < c_cpt)
            def _():
                _ridx(j + 1, (b + 1) % NIB).wait()

            @pl.when(jnp.logical_and(j + 1 < c_cpt, j >= 1))
            def _():
                _scatter_wait(j - 1, (b + 1) % NBUF)

            @pl.when(j + 1 < c_cpt)
            def _():
                _gather((b + 1) % NIB, (b + 1) % NBUF).start()

        return carry

    lax.fori_loop(0, c_cpt // UNROLL, body, 0)

    for b in range(NBUF):
        _scatter_wait(c_cpt - NBUF + b, b)

    plsc.subcore_barrier()

    pltpu.sync_copy(t_sh.at[pl.ds(base, ROWS_PER_TILE)],
                    t_out.at[cid, pl.ds(base, ROWS_PER_TILE)])


def _elu(v):
    return jnp.maximum(v, 0.0) + (jnp.exp(jnp.minimum(v, 0.0)) - 1.0)


def _dense_body(t_ref, deg_ref, xs_ref, w0_ref, b0_ref, l0_ref, lb0_ref,
                w1_ref, b1_ref, l1_ref, lb1_ref, owa_ref, owb_ref, ob_ref,
                out_ref):
    dis = lax.rsqrt(deg_ref[:, 0:1] + deg_ref[:, 1:2] + 1.0)
    agg = (t_ref[0] + t_ref[1] + xs_ref[...]) * dis
    f32 = jnp.float32
    a0 = _elu(jnp.dot(agg, w0_ref[...], preferred_element_type=f32, precision=lax.Precision.HIGHEST) + b0_ref[...])
    y0 = jnp.dot(a0, l0_ref[...], preferred_element_type=f32, precision=lax.Precision.HIGHEST) + lb0_ref[...]
    a1 = _elu(jnp.dot(agg, w1_ref[...], preferred_element_type=f32, precision=lax.Precision.HIGHEST) + b1_ref[...])
    y1 = jnp.dot(a1, l1_ref[...], preferred_element_type=f32, precision=lax.Precision.HIGHEST) + lb1_ref[...]
    out_ref[...] = (jnp.dot(y0, owa_ref[...], preferred_element_type=f32, precision=lax.Precision.HIGHEST)
                    + jnp.dot(y1, owb_ref[...], preferred_element_type=f32, precision=lax.Precision.HIGHEST)
                    + ob_ref[...])


def _dense_call(t2, degT, xs, w0, b0, l0, lb0, w1, b1, l1, lb1, owa, owb, ob):
    blk = 1000

    def full(shape):
        return pl.BlockSpec(shape, lambda i, _s=shape: tuple(0 for _ in _s))

    return pl.pallas_call(
        _dense_body,
        grid=(N // blk,),
        in_specs=[
            pl.BlockSpec((NC, blk, D), lambda i: (0, i, 0)),
            pl.BlockSpec((blk, 2), lambda i: (i, 0)),
            pl.BlockSpec((blk, D), lambda i: (i, 0)),
            full((D, H)), full((1, H)), full((H, H)), full((1, H)),
            full((D, H)), full((1, H)), full((H, H)), full((1, H)),
            full((H, C)), full((H, C)), full((1, C)),
        ],
        out_specs=pl.BlockSpec((blk, C), lambda i: (i, 0)),
        out_shape=jax.ShapeDtypeStruct((N, C), jnp.float32),
    )(t2, degT, xs, w0, b0, l0, lb0, w1, b1, l1, lb1, owa, owb, ob)


def kernel(x, edge_index, conv0_W, conv0_b, lin0_W, lin0_b,
           conv1_W, conv1_b, lin1_W, lin1_b, out_W, out_b):
    pad = EPAD + STAGE_PAD * CHUNK - E
    pad_ar = jnp.arange(pad, dtype=jnp.int32)
    row_p = jnp.concatenate(
        [edge_index[0], pad_ar % N]).reshape(-1, CHUNK)
    col_p = jnp.concatenate(
        [edge_index[1], N + pad_ar % (NPAD - N)]).reshape(-1, CHUNK)

    deg2 = _deg_kernel(col_p, jnp.zeros((NPAD,), jnp.float32)).reshape(NC, NPAD)
    degT = deg2.T

    xs = _xs_call(degT[:N], x)

    t2 = _scatter_kernel(xs, row_p, col_p, jnp.zeros((NPAD, D), jnp.float32))

    return _dense_call(
        t2[:, :N], degT[:N], xs,
        conv0_W, conv0_b.reshape(1, H), lin0_W, lin0_b.reshape(1, H),
        conv1_W, conv1_b.reshape(1, H), lin1_W, lin1_b.reshape(1, H),
        out_W[:H], out_W[H:], out_b.reshape(1, C))


# speedup vs baseline: 2.9819x; 1.2722x over previous
import functools

import jax
import jax.numpy as jnp
from jax import lax
from jax.experimental import pallas as pl
from jax.experimental.pallas import tpu as pltpu
from jax.experimental.pallas import tpu_sc as plsc

N = 10000
D = 128
H = 128
C = 64
E = 320000

NC = 2
NS = 16
NW = NC * NS
CHUNK = 128
CPT = 80
CPT0 = 80
CPT1 = 80
CPTX = max(CPT0, CPT1)
TOTCH = NS * (CPT0 + CPT1)
EPAD = TOTCH * CHUNK
STAGE_PAD = CPTX
ROWS_PER_TILE = 640
NPAD = NS * ROWS_PER_TILE
NBUF = 2
NIB = 4
UNROLL = 4

_mesh = plsc.VectorSubcoreMesh(core_axis_name="c", subcore_axis_name="s")


@functools.partial(
    pl.kernel,
    out_type=jax.ShapeDtypeStruct((NC * NPAD,), jnp.float32),
    mesh=_mesh,
    scratch_types=[
        pltpu.VMEM((CPT, CHUNK), jnp.int32),
        pltpu.VMEM((CHUNK,), jnp.float32),
        pltpu.VMEM_SHARED((NPAD,), jnp.float32),
    ],
)
def _deg_kernel(col_hbm, zeros_hbm, deg_out, col_v, ones_v, deg_sh):
    cid = lax.axis_index("c")
    sid = lax.axis_index("s")
    wid = cid * NS + sid

    base = sid * ROWS_PER_TILE
    pltpu.sync_copy(zeros_hbm.at[pl.ds(base, ROWS_PER_TILE)],
                    deg_sh.at[pl.ds(base, ROWS_PER_TILE)])

    for i in range(CHUNK // 16):
        ones_v[pl.ds(i * 16, 16)] = jnp.ones((16,), jnp.float32)

    pltpu.sync_copy(col_hbm.at[pl.ds(wid * CPT, CPT)], col_v)

    plsc.subcore_barrier()

    def body(j, carry):
        pltpu.sync_copy(ones_v, deg_sh.at[col_v.at[j]], add=True)
        return carry

    lax.fori_loop(0, CPT, body, 0)

    plsc.subcore_barrier()

    pltpu.sync_copy(deg_sh.at[pl.ds(base, ROWS_PER_TILE)],
                    deg_out.at[pl.ds(cid * NPAD + base, ROWS_PER_TILE)])


def _xs_body(deg_ref, x_ref, xs_ref):
    degsum = deg_ref[:, 0:1] + deg_ref[:, 1:2] + 1.0
    xs_ref[...] = x_ref[...] * lax.rsqrt(degsum)


def _xs_call(degT, x):
    blk = 1000
    return pl.pallas_call(
        _xs_body,
        grid=(N // blk,),
        in_specs=[
            pl.BlockSpec((blk, 2), lambda i: (i, 0)),
            pl.BlockSpec((blk, D), lambda i: (i, 0)),
        ],
        out_specs=pl.BlockSpec((blk, D), lambda i: (i, 0)),
        out_shape=jax.ShapeDtypeStruct((N, D), jnp.float32),
    )(degT, x)


@functools.partial(
    pl.kernel,
    out_type=jax.ShapeDtypeStruct((NC, NPAD, D), jnp.float32),
    mesh=_mesh,
    scratch_types=[
        pltpu.VMEM((NIB, CHUNK), jnp.int32),
        pltpu.VMEM((CPTX, CHUNK), jnp.int32),
        pltpu.VMEM((NBUF, CHUNK, D), jnp.float32),
        pltpu.VMEM_SHARED((NPAD, D), jnp.float32),
        pltpu.SemaphoreType.DMA((NIB,)),
        pltpu.SemaphoreType.DMA((NBUF,)),
        pltpu.SemaphoreType.DMA((NBUF,)),
    ],
)
def _scatter_kernel(xs_hbm, row_hbm, col_hbm, zeros_hbm, t_out,
                    row_v, col_v, buf, t_sh, isem, gsem, ssem):
    cid = lax.axis_index("c")
    sid = lax.axis_index("s")
    c_cpt = jnp.where(cid == 0, CPT0, CPT1)
    cbase = jnp.where(cid == 0, sid * CPT0, NS * CPT0 + sid * CPT1)

    base = sid * ROWS_PER_TILE
    pltpu.sync_copy(zeros_hbm.at[pl.ds(base, ROWS_PER_TILE)],
                    t_sh.at[pl.ds(base, ROWS_PER_TILE)])

    pltpu.sync_copy(col_hbm.at[pl.ds(cbase, CPTX)], col_v)

    plsc.subcore_barrier()

    def _ridx(j, s):
        return pltpu.make_async_copy(
            row_hbm.at[cbase + j], row_v.at[s], isem.at[s])

    def _gather(s, b):
        return pltpu.make_async_copy(
            xs_hbm.at[row_v.at[s]], buf.at[b], gsem.at[b])

    def _scatter_start(j, b):
        pltpu.async_copy(buf.at[b], t_sh.at[col_v.at[j]], ssem.at[b],
                         add=True)

    def _scatter_wait(j, b):
        pltpu.make_async_copy(
            buf.at[b], t_sh.at[col_v.at[j]], ssem.at[b]).wait()

    _ridx(0, 0).start()
    _ridx(1, 1).start()
    _ridx(0, 0).wait()
    _gather(0, 0).start()

    def body(i, carry):
        for b in range(UNROLL):
            j = UNROLL * i + b
            bb = b % NBUF
            _gather((b % NIB), bb).wait()
            _scatter_start(j, bb)

            @pl.when(j + 2 < c_cpt)
            def _():
                _ridx(j + 2, (b + 2) % NIB).start()

            @pl.when(j + 1 < c_cpt)
            def _():
                _ridx(j + 1, (b + 1) % NIB).wait()

            @pl.when(jnp.logical_and(j + 1 < c_cpt, j >= 1))
            def _():
                _scatter_wait(j - 1, (b + 1) % NBUF)

            @pl.when(j + 1 < c_cpt)
            def _():
                _gather((b + 1) % NIB, (b + 1) % NBUF).start()

        return carry

    lax.fori_loop(0, c_cpt // UNROLL, body, 0)

    for b in range(NBUF):
        _scatter_wait(c_cpt - NBUF + b, b)

    plsc.subcore_barrier()

    pltpu.sync_copy(t_sh.at[pl.ds(base, ROWS_PER_TILE)],
                    t_out.at[cid, pl.ds(base, ROWS_PER_TILE)])


def _elu(v):
    return jnp.maximum(v, 0.0) + (jnp.exp(jnp.minimum(v, 0.0)) - 1.0)


def _dense_body(t_ref, deg_ref, xs_ref, w0_ref, b0_ref, l0_ref, lb0_ref,
                w1_ref, b1_ref, l1_ref, lb1_ref, owa_ref, owb_ref, ob_ref,
                out_ref):
    dis = lax.rsqrt(deg_ref[:, 0:1] + deg_ref[:, 1:2] + 1.0)
    agg = (t_ref[0] + t_ref[1] + xs_ref[...]) * dis
    f32 = jnp.float32
    a0 = _elu(jnp.dot(agg, w0_ref[...], preferred_element_type=f32) + b0_ref[...])
    y0 = jnp.dot(a0, l0_ref[...], preferred_element_type=f32) + lb0_ref[...]
    a1 = _elu(jnp.dot(agg, w1_ref[...], preferred_element_type=f32) + b1_ref[...])
    y1 = jnp.dot(a1, l1_ref[...], preferred_element_type=f32) + lb1_ref[...]
    out_ref[...] = (jnp.dot(y0, owa_ref[...], preferred_element_type=f32)
                    + jnp.dot(y1, owb_ref[...], preferred_element_type=f32)
                    + ob_ref[...])


def _dense_call(t2, degT, xs, w0, b0, l0, lb0, w1, b1, l1, lb1, owa, owb, ob):
    blk = 1000

    def full(shape):
        return pl.BlockSpec(shape, lambda i, _s=shape: tuple(0 for _ in _s))

    return pl.pallas_call(
        _dense_body,
        grid=(N // blk,),
        in_specs=[
            pl.BlockSpec((NC, blk, D), lambda i: (0, i, 0)),
            pl.BlockSpec((blk, 2), lambda i: (i, 0)),
            pl.BlockSpec((blk, D), lambda i: (i, 0)),
            full((D, H)), full((1, H)), full((H, H)), full((1, H)),
            full((D, H)), full((1, H)), full((H, H)), full((1, H)),
            full((H, C)), full((H, C)), full((1, C)),
        ],
        out_specs=pl.BlockSpec((blk, C), lambda i: (i, 0)),
        out_shape=jax.ShapeDtypeStruct((N, C), jnp.float32),
    )(t2, degT, xs, w0, b0, l0, lb0, w1, b1, l1, lb1, owa, owb, ob)


def kernel(x, edge_index, conv0_W, conv0_b, lin0_W, lin0_b,
           conv1_W, conv1_b, lin1_W, lin1_b, out_W, out_b):
    pad = EPAD + STAGE_PAD * CHUNK - E
    pad_ar = jnp.arange(pad, dtype=jnp.int32)
    row_p = jnp.concatenate(
        [edge_index[0], pad_ar % N]).reshape(-1, CHUNK)
    col_p = jnp.concatenate(
        [edge_index[1], N + pad_ar % (NPAD - N)]).reshape(-1, CHUNK)

    deg2 = _deg_kernel(col_p, jnp.zeros((NPAD,), jnp.float32)).reshape(NC, NPAD)
    degT = deg2.T

    xs = _xs_call(degT, x)

    t2 = _scatter_kernel(xs, row_p, col_p, jnp.zeros((NPAD, D), jnp.float32))

    return _dense_call(
        t2, degT, xs,
        conv0_W, conv0_b.reshape(1, H), lin0_W, lin0_b.reshape(1, H),
        conv1_W, conv1_b.reshape(1, H), lin1_W, lin1_b.reshape(1, H),
        out_W[:H], out_W[H:], out_b.reshape(1, C))


# speedup vs baseline: 3.0881x; 1.0356x over previous
import functools

import jax
import jax.numpy as jnp
from jax import lax
from jax.experimental import pallas as pl
from jax.experimental.pallas import tpu as pltpu
from jax.experimental.pallas import tpu_sc as plsc

N = 10000
D = 128
H = 128
C = 64
E = 320000

NC = 2
NS = 16
NW = NC * NS
CHUNK = 128
CPT = 80
CPT0 = 80
CPT1 = 80
CPTX = max(CPT0, CPT1)
TOTCH = NS * (CPT0 + CPT1)
EPAD = TOTCH * CHUNK
STAGE_PAD = CPTX
ROWS_PER_TILE = 640
NPAD = NS * ROWS_PER_TILE
NBUF = 2
NIB = 4
UNROLL = 4

_mesh = plsc.VectorSubcoreMesh(core_axis_name="c", subcore_axis_name="s")


@functools.partial(
    pl.kernel,
    out_type=jax.ShapeDtypeStruct((NC * NPAD,), jnp.float32),
    mesh=_mesh,
    scratch_types=[
        pltpu.VMEM((CPT, CHUNK), jnp.int32),
        pltpu.VMEM((CHUNK,), jnp.float32),
        pltpu.VMEM((ROWS_PER_TILE,), jnp.float32),
        pltpu.VMEM_SHARED((NPAD,), jnp.float32),
    ],
)
def _deg_kernel(col_hbm, deg_out, col_v, ones_v, zb, deg_sh):
    cid = lax.axis_index("c")
    sid = lax.axis_index("s")
    wid = cid * NS + sid

    base = sid * ROWS_PER_TILE

    def zfill(i, carry):
        zb[pl.ds(i * 16, 16)] = jnp.zeros((16,), jnp.float32)
        return carry

    lax.fori_loop(0, ROWS_PER_TILE // 16, zfill, 0)
    pltpu.sync_copy(zb, deg_sh.at[pl.ds(base, ROWS_PER_TILE)])

    for i in range(CHUNK // 16):
        ones_v[pl.ds(i * 16, 16)] = jnp.ones((16,), jnp.float32)

    pltpu.sync_copy(col_hbm.at[pl.ds(wid * CPT, CPT)], col_v)

    plsc.subcore_barrier()

    def body(j, carry):
        pltpu.sync_copy(ones_v, deg_sh.at[col_v.at[j]], add=True)
        return carry

    lax.fori_loop(0, CPT, body, 0)

    plsc.subcore_barrier()

    pltpu.sync_copy(deg_sh.at[pl.ds(base, ROWS_PER_TILE)],
                    deg_out.at[pl.ds(cid * NPAD + base, ROWS_PER_TILE)])


def _xs_body(deg_ref, x_ref, xs_ref):
    degsum = deg_ref[:, 0:1] + deg_ref[:, 1:2] + 1.0
    xs_ref[...] = x_ref[...] * lax.rsqrt(degsum)


def _xs_call(degT, x):
    blk = 1000
    return pl.pallas_call(
        _xs_body,
        grid=(N // blk,),
        in_specs=[
            pl.BlockSpec((blk, 2), lambda i: (i, 0)),
            pl.BlockSpec((blk, D), lambda i: (i, 0)),
        ],
        out_specs=pl.BlockSpec((blk, D), lambda i: (i, 0)),
        out_shape=jax.ShapeDtypeStruct((N, D), jnp.float32),
    )(degT, x)


@functools.partial(
    pl.kernel,
    out_type=jax.ShapeDtypeStruct((NC, NPAD, D), jnp.float32),
    mesh=_mesh,
    scratch_types=[
        pltpu.VMEM((NIB, CHUNK), jnp.int32),
        pltpu.VMEM((CPTX, CHUNK), jnp.int32),
        pltpu.VMEM((NBUF, CHUNK, D), jnp.float32),
        pltpu.VMEM_SHARED((NPAD, D), jnp.float32),
        pltpu.SemaphoreType.DMA((NIB,)),
        pltpu.SemaphoreType.DMA((NBUF,)),
        pltpu.SemaphoreType.DMA((NBUF,)),
    ],
)
def _scatter_kernel(xs_hbm, row_hbm, col_hbm, t_out,
                    row_v, col_v, buf, t_sh, isem, gsem, ssem):
    cid = lax.axis_index("c")
    sid = lax.axis_index("s")
    c_cpt = jnp.where(cid == 0, CPT0, CPT1)
    cbase = jnp.where(cid == 0, sid * CPT0, NS * CPT0 + sid * CPT1)

    base = sid * ROWS_PER_TILE

    def zfill(i, carry):
        for v in range(CHUNK // 16):
            buf[0, i, pl.ds(v * 16, 16)] = jnp.zeros((16,), jnp.float32)
        return carry

    lax.fori_loop(0, CHUNK, zfill, 0)
    for r in range(ROWS_PER_TILE // CHUNK):
        pltpu.sync_copy(buf.at[0],
                        t_sh.at[pl.ds(base + r * CHUNK, CHUNK)])

    pltpu.sync_copy(col_hbm.at[pl.ds(cbase, CPTX)], col_v)

    plsc.subcore_barrier()

    def _ridx(j, s):
        return pltpu.make_async_copy(
            row_hbm.at[cbase + j], row_v.at[s], isem.at[s])

    def _gather(s, b):
        return pltpu.make_async_copy(
            xs_hbm.at[row_v.at[s]], buf.at[b], gsem.at[b])

    def _scatter_start(j, b):
        pltpu.async_copy(buf.at[b], t_sh.at[col_v.at[j]], ssem.at[b],
                         add=True)

    def _scatter_wait(j, b):
        pltpu.make_async_copy(
            buf.at[b], t_sh.at[col_v.at[j]], ssem.at[b]).wait()

    _ridx(0, 0).start()
    _ridx(1, 1).start()
    _ridx(0, 0).wait()
    _gather(0, 0).start()

    def body(i, carry):
        for b in range(UNROLL):
            j = UNROLL * i + b
            bb = b % NBUF
            _gather((b % NIB), bb).wait()
            _scatter_start(j, bb)

            @pl.when(j + 2 < c_cpt)
            def _():
                _ridx(j + 2, (b + 2) % NIB).start()

            @pl.when(j + 1 < c_cpt)
            def _():
                _ridx(j + 1, (b + 1) % NIB).wait()

            @pl.when(jnp.logical_and(j + 1 < c_cpt, j >= 1))
            def _():
                _scatter_wait(j - 1, (b + 1) % NBUF)

            @pl.when(j + 1 < c_cpt)
            def _():
                _gather((b + 1) % NIB, (b + 1) % NBUF).start()

        return carry

    lax.fori_loop(0, c_cpt // UNROLL, body, 0)

    for b in range(NBUF):
        _scatter_wait(c_cpt - NBUF + b, b)

    plsc.subcore_barrier()

    pltpu.sync_copy(t_sh.at[pl.ds(base, ROWS_PER_TILE)],
                    t_out.at[cid, pl.ds(base, ROWS_PER_TILE)])


def _elu(v):
    return jnp.maximum(v, 0.0) + (jnp.exp(jnp.minimum(v, 0.0)) - 1.0)


def _dense_body(t_ref, deg_ref, xs_ref, w0_ref, b0_ref, l0_ref, lb0_ref,
                w1_ref, b1_ref, l1_ref, lb1_ref, owa_ref, owb_ref, ob_ref,
                out_ref):
    dis = lax.rsqrt(deg_ref[:, 0:1] + deg_ref[:, 1:2] + 1.0)
    agg = (t_ref[0] + t_ref[1] + xs_ref[...]) * dis
    f32 = jnp.float32
    a0 = _elu(jnp.dot(agg, w0_ref[...], preferred_element_type=f32) + b0_ref[...])
    y0 = jnp.dot(a0, l0_ref[...], preferred_element_type=f32) + lb0_ref[...]
    a1 = _elu(jnp.dot(agg, w1_ref[...], preferred_element_type=f32) + b1_ref[...])
    y1 = jnp.dot(a1, l1_ref[...], preferred_element_type=f32) + lb1_ref[...]
    out_ref[...] = (jnp.dot(y0, owa_ref[...], preferred_element_type=f32)
                    + jnp.dot(y1, owb_ref[...], preferred_element_type=f32)
                    + ob_ref[...])


def _dense_call(t2, degT, xs, w0, b0, l0, lb0, w1, b1, l1, lb1, owa, owb, ob):
    blk = 1000

    def full(shape):
        return pl.BlockSpec(shape, lambda i, _s=shape: tuple(0 for _ in _s))

    return pl.pallas_call(
        _dense_body,
        grid=(N // blk,),
        in_specs=[
            pl.BlockSpec((NC, blk, D), lambda i: (0, i, 0)),
            pl.BlockSpec((blk, 2), lambda i: (i, 0)),
            pl.BlockSpec((blk, D), lambda i: (i, 0)),
            full((D, H)), full((1, H)), full((H, H)), full((1, H)),
            full((D, H)), full((1, H)), full((H, H)), full((1, H)),
            full((H, C)), full((H, C)), full((1, C)),
        ],
        out_specs=pl.BlockSpec((blk, C), lambda i: (i, 0)),
        out_shape=jax.ShapeDtypeStruct((N, C), jnp.float32),
    )(t2, degT, xs, w0, b0, l0, lb0, w1, b1, l1, lb1, owa, owb, ob)


def kernel(x, edge_index, conv0_W, conv0_b, lin0_W, lin0_b,
           conv1_W, conv1_b, lin1_W, lin1_b, out_W, out_b):
    pad = EPAD + STAGE_PAD * CHUNK - E
    pad_ar = jnp.arange(pad, dtype=jnp.int32)
    row_p = jnp.concatenate(
        [edge_index[0], pad_ar % N]).reshape(-1, CHUNK)
    col_p = jnp.concatenate(
        [edge_index[1], N + pad_ar % (NPAD - N)]).reshape(-1, CHUNK)

    deg2 = _deg_kernel(col_p).reshape(NC, NPAD)
    degT = deg2.T

    xs = _xs_call(degT, x)

    t2 = _scatter_kernel(xs, row_p, col_p)

    return _dense_call(
        t2, degT, xs,
        conv0_W, conv0_b.reshape(1, H), lin0_W, lin0_b.reshape(1, H),
        conv1_W, conv1_b.reshape(1, H), lin1_W, lin1_b.reshape(1, H),
        out_W[:H], out_W[H:], out_b.reshape(1, C))


# speedup vs baseline: 3.2441x; 1.0505x over previous
import functools

import jax
import jax.numpy as jnp
from jax import lax
from jax.experimental import pallas as pl
from jax.experimental.pallas import tpu as pltpu
from jax.experimental.pallas import tpu_sc as plsc

N = 10000
D = 128
H = 128
C = 64
E = 320000

NC = 2
NS = 16
NW = NC * NS
CHUNK = 128
CPT = 80
CPT0 = 80
CPT1 = 80
CPTX = max(CPT0, CPT1)
TOTCH = NS * (CPT0 + CPT1)
EPAD = TOTCH * CHUNK
STAGE_PAD = CPTX
ROWS_PER_TILE = 640
NPAD = NS * ROWS_PER_TILE
NBUF = 2
NIB = 4
UNROLL = 4

_mesh = plsc.VectorSubcoreMesh(core_axis_name="c", subcore_axis_name="s")


@functools.partial(
    pl.kernel,
    out_type=jax.ShapeDtypeStruct((NC * NPAD,), jnp.float32),
    mesh=_mesh,
    scratch_types=[
        pltpu.VMEM((CPT, CHUNK), jnp.int32),
        pltpu.VMEM((CHUNK,), jnp.float32),
        pltpu.VMEM((ROWS_PER_TILE,), jnp.float32),
        pltpu.VMEM_SHARED((NPAD,), jnp.float32),
        pltpu.SemaphoreType.DMA,
    ],
)
def _deg_kernel(col_hbm, deg_out, col_v, ones_v, zb, deg_sh, dsem):
    cid = lax.axis_index("c")
    sid = lax.axis_index("s")
    wid = cid * NS + sid

    base = sid * ROWS_PER_TILE

    def zfill(i, carry):
        zb[pl.ds(i * 16, 16)] = jnp.zeros((16,), jnp.float32)
        return carry

    lax.fori_loop(0, ROWS_PER_TILE // 16, zfill, 0)
    pltpu.sync_copy(zb, deg_sh.at[pl.ds(base, ROWS_PER_TILE)])

    for i in range(CHUNK // 16):
        ones_v[pl.ds(i * 16, 16)] = jnp.ones((16,), jnp.float32)

    pltpu.sync_copy(col_hbm.at[pl.ds(wid * CPT, CPT)], col_v)

    plsc.subcore_barrier()

    WIN = 8

    def body(j, carry):
        pltpu.async_copy(ones_v, deg_sh.at[col_v.at[j]], dsem, add=True)

        @pl.when(j >= WIN)
        def _():
            pltpu.make_async_copy(ones_v, deg_sh.at[col_v.at[j]], dsem).wait()

        return carry

    lax.fori_loop(0, CPT, body, 0)
    for _ in range(WIN):
        pltpu.make_async_copy(ones_v, deg_sh.at[col_v.at[0]], dsem).wait()

    plsc.subcore_barrier()

    pltpu.sync_copy(deg_sh.at[pl.ds(base, ROWS_PER_TILE)],
                    deg_out.at[pl.ds(cid * NPAD + base, ROWS_PER_TILE)])


def _xs_body(deg_ref, x_ref, xs_ref):
    degsum = deg_ref[:, 0:1] + deg_ref[:, 1:2] + 1.0
    xs_ref[...] = x_ref[...] * lax.rsqrt(degsum)


def _xs_call(degT, x):
    blk = 2000
    return pl.pallas_call(
        _xs_body,
        grid=(N // blk,),
        in_specs=[
            pl.BlockSpec((blk, 2), lambda i: (i, 0)),
            pl.BlockSpec((blk, D), lambda i: (i, 0)),
        ],
        out_specs=pl.BlockSpec((blk, D), lambda i: (i, 0)),
        out_shape=jax.ShapeDtypeStruct((N, D), jnp.float32),
    )(degT, x)


@functools.partial(
    pl.kernel,
    out_type=jax.ShapeDtypeStruct((NC, NPAD, D), jnp.float32),
    mesh=_mesh,
    scratch_types=[
        pltpu.VMEM((NIB, CHUNK), jnp.int32),
        pltpu.VMEM((CPTX, CHUNK), jnp.int32),
        pltpu.VMEM((NBUF, CHUNK, D), jnp.float32),
        pltpu.VMEM_SHARED((NPAD, D), jnp.float32),
        pltpu.SemaphoreType.DMA((NIB,)),
        pltpu.SemaphoreType.DMA((NBUF,)),
        pltpu.SemaphoreType.DMA((NBUF,)),
    ],
)
def _scatter_kernel(xs_hbm, row_hbm, col_hbm, t_out,
                    row_v, col_v, buf, t_sh, isem, gsem, ssem):
    cid = lax.axis_index("c")
    sid = lax.axis_index("s")
    c_cpt = jnp.where(cid == 0, CPT0, CPT1)
    cbase = jnp.where(cid == 0, sid * CPT0, NS * CPT0 + sid * CPT1)

    base = sid * ROWS_PER_TILE

    def zfill(i, carry):
        for v in range(CHUNK // 16):
            buf[0, i, pl.ds(v * 16, 16)] = jnp.zeros((16,), jnp.float32)
        return carry

    lax.fori_loop(0, CHUNK, zfill, 0)
    for r in range(ROWS_PER_TILE // CHUNK):
        pltpu.sync_copy(buf.at[0],
                        t_sh.at[pl.ds(base + r * CHUNK, CHUNK)])

    pltpu.sync_copy(col_hbm.at[pl.ds(cbase, CPTX)], col_v)

    plsc.subcore_barrier()

    def _ridx(j, s):
        return pltpu.make_async_copy(
            row_hbm.at[cbase + j], row_v.at[s], isem.at[s])

    def _gather(s, b):
        return pltpu.make_async_copy(
            xs_hbm.at[row_v.at[s]], buf.at[b], gsem.at[b])

    def _scatter_start(j, b):
        pltpu.async_copy(buf.at[b], t_sh.at[col_v.at[j]], ssem.at[b],
                         add=True)

    def _scatter_wait(j, b):
        pltpu.make_async_copy(
            buf.at[b], t_sh.at[col_v.at[j]], ssem.at[b]).wait()

    _ridx(0, 0).start()
    _ridx(1, 1).start()
    _ridx(0, 0).wait()
    _gather(0, 0).start()

    def body(i, carry):
        for b in range(UNROLL):
            j = UNROLL * i + b
            bb = b % NBUF
            _gather((b % NIB), bb).wait()
            _scatter_start(j, bb)

            @pl.when(j + 2 < c_cpt)
            def _():
                _ridx(j + 2, (b + 2) % NIB).start()

            @pl.when(j + 1 < c_cpt)
            def _():
                _ridx(j + 1, (b + 1) % NIB).wait()

            @pl.when(jnp.logical_and(j + 1 < c_cpt, j >= 1))
            def _():
                _scatter_wait(j - 1, (b + 1) % NBUF)

            @pl.when(j + 1 < c_cpt)
            def _():
                _gather((b + 1) % NIB, (b + 1) % NBUF).start()

        return carry

    lax.fori_loop(0, c_cpt // UNROLL, body, 0)

    for b in range(NBUF):
        _scatter_wait(c_cpt - NBUF + b, b)

    plsc.subcore_barrier()

    pltpu.sync_copy(t_sh.at[pl.ds(base, ROWS_PER_TILE)],
                    t_out.at[cid, pl.ds(base, ROWS_PER_TILE)])


def _elu(v):
    return jnp.maximum(v, 0.0) + (jnp.exp(jnp.minimum(v, 0.0)) - 1.0)


def _dense_body(t_ref, deg_ref, xs_ref, w0_ref, b0_ref, l0_ref, lb0_ref,
                w1_ref, b1_ref, l1_ref, lb1_ref, owa_ref, owb_ref, ob_ref,
                out_ref):
    dis = lax.rsqrt(deg_ref[:, 0:1] + deg_ref[:, 1:2] + 1.0)
    agg = (t_ref[0] + t_ref[1] + xs_ref[...]) * dis
    f32 = jnp.float32
    a0 = _elu(jnp.dot(agg, w0_ref[...], preferred_element_type=f32) + b0_ref[...])
    y0 = jnp.dot(a0, l0_ref[...], preferred_element_type=f32) + lb0_ref[...]
    a1 = _elu(jnp.dot(agg, w1_ref[...], preferred_element_type=f32) + b1_ref[...])
    y1 = jnp.dot(a1, l1_ref[...], preferred_element_type=f32) + lb1_ref[...]
    out_ref[...] = (jnp.dot(y0, owa_ref[...], preferred_element_type=f32)
                    + jnp.dot(y1, owb_ref[...], preferred_element_type=f32)
                    + ob_ref[...])


def _dense_call(t2, degT, xs, w0, b0, l0, lb0, w1, b1, l1, lb1, owa, owb, ob):
    blk = 2000

    def full(shape):
        return pl.BlockSpec(shape, lambda i, _s=shape: tuple(0 for _ in _s))

    return pl.pallas_call(
        _dense_body,
        grid=(N // blk,),
        in_specs=[
            pl.BlockSpec((NC, blk, D), lambda i: (0, i, 0)),
            pl.BlockSpec((blk, 2), lambda i: (i, 0)),
            pl.BlockSpec((blk, D), lambda i: (i, 0)),
            full((D, H)), full((1, H)), full((H, H)), full((1, H)),
            full((D, H)), full((1, H)), full((H, H)), full((1, H)),
            full((H, C)), full((H, C)), full((1, C)),
        ],
        out_specs=pl.BlockSpec((blk, C), lambda i: (i, 0)),
        out_shape=jax.ShapeDtypeStruct((N, C), jnp.float32),
    )(t2, degT, xs, w0, b0, l0, lb0, w1, b1, l1, lb1, owa, owb, ob)


def kernel(x, edge_index, conv0_W, conv0_b, lin0_W, lin0_b,
           conv1_W, conv1_b, lin1_W, lin1_b, out_W, out_b):
    pad = EPAD + STAGE_PAD * CHUNK - E
    pad_ar = jnp.arange(pad, dtype=jnp.int32)
    col_p = jnp.concatenate(
        [edge_index[1], N + pad_ar % (NPAD - N)]).reshape(-1, CHUNK)

    deg2 = _deg_kernel(col_p).reshape(NC, NPAD)
    degT = deg2.T

    row_p = jnp.concatenate(
        [edge_index[0], pad_ar % N]).reshape(-1, CHUNK)

    xs = _xs_call(degT, x)

    t2 = _scatter_kernel(xs, row_p, col_p)

    return _dense_call(
        t2, degT, xs,
        conv0_W, conv0_b.reshape(1, H), lin0_W, lin0_b.reshape(1, H),
        conv1_W, conv1_b.reshape(1, H), lin1_W, lin1_b.reshape(1, H),
        out_W[:H], out_W[H:], out_b.reshape(1, C))
